# baseline (speedup 1.0000x reference)
import functools

import jax
import jax.numpy as jnp
from jax import lax
from jax.experimental import pallas as pl
from jax.experimental.pallas import tpu as pltpu


def _router_body(x_hbm, w1_ref, b1_ref, w2_ref, b2_ref, out_ref, idx_ref,
                 h_ref, xstage_ref, xb_ref, dma_sem, *,
                 k_top, n_e, bn, bh, prec1, prec2):
    i = pl.program_id(0)
    j = pl.program_id(1)
    ni = pl.num_programs(0)
    nj = pl.num_programs(1)
    slot = lax.rem(i, 2)
    nxt = lax.rem(i + 1, 2)

    @pl.when(jnp.logical_and(i == 0, j == 0))
    def _startup():
        cp = pltpu.make_async_copy(
            x_hbm.at[pl.ds(0, bn), :], xstage_ref, dma_sem)
        cp.start()
        cp.wait()
        xb_ref[0] = xstage_ref[...].astype(jnp.bfloat16)

    @pl.when(jnp.logical_and(j == 1, i + 1 < ni))
    def _prefetch_next():
        pltpu.make_async_copy(
            x_hbm.at[pl.ds((i + 1) * bn, bn), :], xstage_ref,
            dma_sem).start()

    @pl.when(jnp.logical_and(j == nj - 2, i + 1 < ni))
    def _cast_next():
        pltpu.make_async_copy(
            x_hbm.at[pl.ds((i + 1) * bn, bn), :], xstage_ref,
            dma_sem).wait()
        xb_ref[nxt] = xstage_ref[...].astype(jnp.bfloat16)

    w1b = w1_ref[...].astype(jnp.bfloat16)
    hp = lax.dot_general(xb_ref[slot], w1b, (((1,), (1,)), ((), ())),
                         preferred_element_type=jnp.float32, precision=prec1)
    hp = jnp.maximum(hp + b1_ref[...], 0.0)
    h_ref[:, pl.ds(j * bh, bh)] = hp.astype(jnp.bfloat16)

    @pl.when(j == nj - 1)
    def _epilogue():
        logits = lax.dot_general(
            h_ref[...], w2_ref[...], (((1,), (1,)), ((), ())),
            preferred_element_type=jnp.float32, precision=prec2) + b2_ref[...]
        e_iota = lax.broadcasted_iota(jnp.int32, (bn, n_e), 1)
        r_iota = lax.broadcasted_iota(jnp.int32, (bn, k_top), 1)
        work = logits
        sel = jnp.zeros((bn, n_e), jnp.bool_)
        idx_out = jnp.zeros((bn, k_top), jnp.int32)
        top0 = None
        for k in range(k_top):
            m = jnp.max(work, axis=1, keepdims=True)
            hit = work == m
            idxk = jnp.min(jnp.where(hit, e_iota, n_e), axis=1, keepdims=True)
            picked = e_iota == idxk
            work = jnp.where(picked, -jnp.inf, work)
            sel = jnp.logical_or(sel, picked)
            idx_out = jnp.where(r_iota == k, idxk, idx_out)
            if k == 0:
                top0 = m
        ex = jnp.where(sel, jnp.exp(logits - top0), 0.0)
        out_ref[...] = ex / jnp.sum(ex, axis=1, keepdims=True)
        idx_ref[...] = idx_out


@jax.jit
def kernel(x, W1, b1, W2, b2):
    n, d = x.shape
    h_dim = W1.shape[0]
    n_e = W2.shape[0]
    k_top = 8
    bn = min(512, n)
    bh = min(512, h_dim)
    assert n % bn == 0 and h_dim % bh == 0

    b1r = b1.reshape(1, h_dim)
    b2r = b2.reshape(1, n_e)
    w2b = W2.astype(jnp.bfloat16)

    body = functools.partial(
        _router_body, k_top=k_top, n_e=n_e, bn=bn, bh=bh,
        prec1=lax.Precision.DEFAULT, prec2=lax.Precision.DEFAULT)

    out, idx = pl.pallas_call(
        body,
        grid=(n // bn, h_dim // bh),
        in_specs=[
            pl.BlockSpec(memory_space=pl.ANY),
            pl.BlockSpec((bh, d), lambda i, j: (j, 0)),
            pl.BlockSpec((1, bh), lambda i, j: (0, j)),
            pl.BlockSpec((n_e, h_dim), lambda i, j: (0, 0)),
            pl.BlockSpec((1, n_e), lambda i, j: (0, 0)),
        ],
        out_specs=[
            pl.BlockSpec((bn, n_e), lambda i, j: (i, 0)),
            pl.BlockSpec((bn, k_top), lambda i, j: (i, 0)),
        ],
        out_shape=[
            jax.ShapeDtypeStruct((n, n_e), jnp.float32),
            jax.ShapeDtypeStruct((n, k_top), jnp.int32),
        ],
        scratch_shapes=[pltpu.VMEM((bn, h_dim), jnp.bfloat16),
                        pltpu.VMEM((bn, d), jnp.float32),
                        pltpu.VMEM((2, bn, d), jnp.bfloat16),
                        pltpu.SemaphoreType.DMA],
        compiler_params=pltpu.CompilerParams(
            dimension_semantics=("arbitrary", "arbitrary")),
    )(x, W1, b1r, w2b, b2r)
    return (out, idx)


# speedup vs baseline: 1.0781x; 1.0781x over previous
import functools

import jax
import jax.numpy as jnp
from jax import lax
from jax.experimental import pallas as pl
from jax.experimental.pallas import tpu as pltpu


def _router_body(x_ref, w1_ref, b1_ref, w2_ref, b2_ref, out_ref, idx_ref,
                 h_ref, *, k_top, n_e, bn, bh, prec1, prec2):
    j = pl.program_id(1)
    nj = pl.num_programs(1)

    hp = lax.dot_general(x_ref[...], w1_ref[...], (((1,), (1,)), ((), ())),
                         preferred_element_type=jnp.float32, precision=prec1)
    hp = jnp.maximum(hp + b1_ref[...], 0.0)
    h_ref[:, pl.ds(j * bh, bh)] = hp.astype(jnp.bfloat16)

    @pl.when(j == nj - 1)
    def _epilogue():
        logits = lax.dot_general(
            h_ref[...], w2_ref[...], (((1,), (1,)), ((), ())),
            preferred_element_type=jnp.float32, precision=prec2) + b2_ref[...]
        e_iota = lax.broadcasted_iota(jnp.int32, (bn, n_e), 1)
        r_iota = lax.broadcasted_iota(jnp.int32, (bn, k_top), 1)
        work = logits
        sel = jnp.zeros((bn, n_e), jnp.bool_)
        idx_out = jnp.zeros((bn, k_top), jnp.int32)
        top0 = None
        for k in range(k_top):
            m = jnp.max(work, axis=1, keepdims=True)
            hit = work == m
            idxk = jnp.min(jnp.where(hit, e_iota, n_e), axis=1, keepdims=True)
            picked = e_iota == idxk
            work = jnp.where(picked, -jnp.inf, work)
            sel = jnp.logical_or(sel, picked)
            idx_out = jnp.where(r_iota == k, idxk, idx_out)
            if k == 0:
                top0 = m
        ex = jnp.where(sel, jnp.exp(logits - top0), 0.0)
        out_ref[...] = ex / jnp.sum(ex, axis=1, keepdims=True)
        idx_ref[...] = idx_out


@jax.jit
def kernel(x, W1, b1, W2, b2):
    n, d = x.shape
    h_dim = W1.shape[0]
    n_e = W2.shape[0]
    k_top = 8
    bn = min(1024, n)
    bh = min(512, h_dim)
    assert n % bn == 0 and h_dim % bh == 0

    xb = x.astype(jnp.bfloat16)
    w1b = W1.astype(jnp.bfloat16)
    w2b = W2.astype(jnp.bfloat16)
    b1r = b1.reshape(1, h_dim)
    b2r = b2.reshape(1, n_e)

    body = functools.partial(
        _router_body, k_top=k_top, n_e=n_e, bn=bn, bh=bh,
        prec1=lax.Precision.DEFAULT, prec2=lax.Precision.DEFAULT)

    out, idx = pl.pallas_call(
        body,
        grid=(n // bn, h_dim // bh),
        in_specs=[
            pl.BlockSpec((bn, d), lambda i, j: (i, 0)),
            pl.BlockSpec((bh, d), lambda i, j: (j, 0)),
            pl.BlockSpec((1, bh), lambda i, j: (0, j)),
            pl.BlockSpec((n_e, h_dim), lambda i, j: (0, 0)),
            pl.BlockSpec((1, n_e), lambda i, j: (0, 0)),
        ],
        out_specs=[
            pl.BlockSpec((bn, n_e), lambda i, j: (i, 0)),
            pl.BlockSpec((bn, k_top), lambda i, j: (i, 0)),
        ],
        out_shape=[
            jax.ShapeDtypeStruct((n, n_e), jnp.float32),
            jax.ShapeDtypeStruct((n, k_top), jnp.int32),
        ],
        scratch_shapes=[pltpu.VMEM((bn, h_dim), jnp.bfloat16)],
        compiler_params=pltpu.CompilerParams(
            dimension_semantics=("arbitrary", "arbitrary")),
    )(xb, w1b, b1r, w2b, b2r)
    return (out, idx)


# speedup vs baseline: 1.2564x; 1.1654x over previous
import functools

import jax
import jax.numpy as jnp
from jax import lax
from jax.experimental import pallas as pl
from jax.experimental.pallas import tpu as pltpu


def _router_body(x_ref, w1_ref, b1_ref, w2_ref, b2_ref, out_ref, idx_ref,
                 *, k_top, n_e, bn, prec1, prec2):
    xb = x_ref[...].astype(jnp.bfloat16)
    hp = lax.dot_general(xb, w1_ref[...], (((1,), (1,)), ((), ())),
                         preferred_element_type=jnp.float32, precision=prec1)
    hb = jnp.maximum(hp + b1_ref[...], 0.0).astype(jnp.bfloat16)
    logits = lax.dot_general(
        hb, w2_ref[...], (((1,), (1,)), ((), ())),
        preferred_element_type=jnp.float32, precision=prec2) + b2_ref[...]
    e_iota = lax.broadcasted_iota(jnp.int32, (bn, n_e), 1)
    r_iota = lax.broadcasted_iota(jnp.int32, (bn, k_top), 1)
    work = logits
    sel = jnp.zeros((bn, n_e), jnp.bool_)
    idx_out = jnp.zeros((bn, k_top), jnp.int32)
    top0 = None
    for k in range(k_top):
        m = jnp.max(work, axis=1, keepdims=True)
        hit = work == m
        idxk = jnp.min(jnp.where(hit, e_iota, n_e), axis=1, keepdims=True)
        picked = e_iota == idxk
        work = jnp.where(picked, -jnp.inf, work)
        sel = jnp.logical_or(sel, picked)
        idx_out = jnp.where(r_iota == k, idxk, idx_out)
        if k == 0:
            top0 = m
    ex = jnp.where(sel, jnp.exp(logits - top0), 0.0)
    out_ref[...] = ex / jnp.sum(ex, axis=1, keepdims=True)
    idx_ref[...] = idx_out


@jax.jit
def kernel(x, W1, b1, W2, b2):
    n, d = x.shape
    h_dim = W1.shape[0]
    n_e = W2.shape[0]
    k_top = 8
    bn = min(512, n)
    assert n % bn == 0

    w1b = W1.astype(jnp.bfloat16)
    w2b = W2.astype(jnp.bfloat16)
    b1r = b1.reshape(1, h_dim)
    b2r = b2.reshape(1, n_e)

    body = functools.partial(
        _router_body, k_top=k_top, n_e=n_e, bn=bn,
        prec1=lax.Precision.DEFAULT, prec2=lax.Precision.DEFAULT)

    out, idx = pl.pallas_call(
        body,
        grid=(n // bn,),
        in_specs=[
            pl.BlockSpec((bn, d), lambda i: (i, 0)),
            pl.BlockSpec((h_dim, d), lambda i: (0, 0)),
            pl.BlockSpec((1, h_dim), lambda i: (0, 0)),
            pl.BlockSpec((n_e, h_dim), lambda i: (0, 0)),
            pl.BlockSpec((1, n_e), lambda i: (0, 0)),
        ],
        out_specs=[
            pl.BlockSpec((bn, n_e), lambda i: (i, 0)),
            pl.BlockSpec((bn, k_top), lambda i: (i, 0)),
        ],
        out_shape=[
            jax.ShapeDtypeStruct((n, n_e), jnp.float32),
            jax.ShapeDtypeStruct((n, k_top), jnp.int32),
        ],
        compiler_params=pltpu.CompilerParams(
            dimension_semantics=("arbitrary",)),
    )(x, w1b, b1r, w2b, b2r)
    return (out, idx)


# speedup vs baseline: 1.2986x; 1.0335x over previous
import functools

import jax
import jax.numpy as jnp
from jax import lax
from jax.experimental import pallas as pl
from jax.experimental.pallas import tpu as pltpu


def _router_body(x_ref, w1_ref, b1_ref, w2_ref, b2_ref, out_ref, idx_ref,
                 *, k_top, n_e, bn, prec1, prec2):
    xb = x_ref[...].astype(jnp.bfloat16)
    hp = lax.dot_general(xb, w1_ref[...], (((1,), (1,)), ((), ())),
                         preferred_element_type=jnp.float32, precision=prec1)
    hb = jnp.maximum(hp + b1_ref[...], 0.0).astype(jnp.bfloat16)
    logits = lax.dot_general(
        hb, w2_ref[...], (((1,), (1,)), ((), ())),
        preferred_element_type=jnp.float32, precision=prec2) + b2_ref[...]
    e_iota = lax.broadcasted_iota(
        jnp.int32, (bn, n_e), 1).astype(jnp.float32)
    r_iota = lax.broadcasted_iota(jnp.int32, (bn, k_top), 1)
    work = logits
    sel = jnp.zeros((bn, n_e), jnp.bool_)
    idx_out = jnp.zeros((bn, k_top), jnp.int32)
    top0 = None
    for k in range(k_top):
        m = jnp.max(work, axis=1, keepdims=True)
        hit = work == m
        idxk = jnp.min(jnp.where(hit, e_iota, float(n_e)), axis=1,
                       keepdims=True)
        picked = e_iota == idxk
        work = jnp.where(picked, -jnp.inf, work)
        sel = jnp.logical_or(sel, picked)
        idx_out = jnp.where(r_iota == k, idxk.astype(jnp.int32), idx_out)
        if k == 0:
            top0 = m
    ex = jnp.where(sel, jnp.exp(logits - top0), 0.0)
    out_ref[...] = ex / jnp.sum(ex, axis=1, keepdims=True)
    idx_ref[...] = idx_out


@jax.jit
def kernel(x, W1, b1, W2, b2):
    n, d = x.shape
    h_dim = W1.shape[0]
    n_e = W2.shape[0]
    k_top = 8
    bn = min(512, n)
    assert n % bn == 0

    w1b = W1.astype(jnp.bfloat16)
    w2b = W2.astype(jnp.bfloat16)
    b1r = b1.reshape(1, h_dim)
    b2r = b2.reshape(1, n_e)

    body = functools.partial(
        _router_body, k_top=k_top, n_e=n_e, bn=bn,
        prec1=lax.Precision.DEFAULT, prec2=lax.Precision.DEFAULT)

    out, idx = pl.pallas_call(
        body,
        grid=(n // bn,),
        in_specs=[
            pl.BlockSpec((bn, d), lambda i: (i, 0)),
            pl.BlockSpec((h_dim, d), lambda i: (0, 0)),
            pl.BlockSpec((1, h_dim), lambda i: (0, 0)),
            pl.BlockSpec((n_e, h_dim), lambda i: (0, 0)),
            pl.BlockSpec((1, n_e), lambda i: (0, 0)),
        ],
        out_specs=[
            pl.BlockSpec((bn, n_e), lambda i: (i, 0)),
            pl.BlockSpec((bn, k_top), lambda i: (i, 0)),
        ],
        out_shape=[
            jax.ShapeDtypeStruct((n, n_e), jnp.float32),
            jax.ShapeDtypeStruct((n, k_top), jnp.int32),
        ],
        compiler_params=pltpu.CompilerParams(
            dimension_semantics=("arbitrary",)),
    )(x, w1b, b1r, w2b, b2r)
    return (out, idx)
